# baseline (speedup 1.0000x reference)
import functools

import jax
import jax.numpy as jnp
import numpy as np
from jax import lax
from jax.experimental import pallas as pl
from jax.experimental.pallas import tpu as pltpu
from jax.experimental.pallas import tpu_sc as plsc

SQRT3 = np.float32(3 ** 0.5)
HALF = np.float32(0.5)
MAGIC = np.float32(12582912.0)

NUM_WORKERS = 32
ROUNDS = 4


def _rne(v):
    return (v + MAGIC) - MAGIC


def _compute(src, dst, nvec):
    @plsc.parallel_loop(0, nvec, 1, unroll=6)
    def vec_body(v):
        off = lax.shift_left(v >> 3, 8) + lax.shift_left(v & 7, 4)
        a0 = src[pl.ds(off, 16)]
        a1 = src[pl.ds(off + 128, 16)]
        s1 = a1 / SQRT3
        r10 = _rne(a0)
        r11 = _rne(s1)
        r20 = _rne(a0 - HALF) + HALF
        r21 = _rne(s1 - HALF) + HALF
        y11 = r11 * SQRT3
        y21 = r21 * SQRT3
        e10 = a0 - r10
        e11 = a1 - y11
        e20 = a0 - r20
        e21 = a1 - y21
        d1 = e10 * e10 + e11 * e11
        d2 = e20 * e20 + e21 * e21
        take = d1 <= d2
        dst[pl.ds(off, 16)] = jnp.where(take, r10, r20)
        dst[pl.ds(off + 128, 16)] = jnp.where(take, y11, y21)


def _sc_body(x_hbm, o_hbm, in0, in1, out0, out1, sems, chunk):
    wid = lax.axis_index("c") * 16 + lax.axis_index("s")
    base = wid * chunk
    sub = chunk // ROUNDS
    nvec = sub // 32
    ins = (in0, in1)
    outs = (out0, out1)

    in_copies = [None] * ROUNDS
    out_copies = [None] * ROUNDS
    in_copies[0] = pltpu.async_copy(x_hbm.at[pl.ds(base, sub)], ins[0], sems.at[0])
    for r in range(ROUNDS):
        b = r % 2
        if r + 1 < ROUNDS:
            in_copies[r + 1] = pltpu.async_copy(
                x_hbm.at[pl.ds(base + (r + 1) * sub, sub)], ins[1 - b], sems.at[1 - b]
            )
        in_copies[r].wait()
        if r >= 2:
            out_copies[r - 2].wait()
        _compute(ins[b], outs[b], nvec)
        out_copies[r] = pltpu.async_copy(
            outs[b], o_hbm.at[pl.ds(base + r * sub, sub)], sems.at[2 + b]
        )
    out_copies[ROUNDS - 2].wait()
    out_copies[ROUNDS - 1].wait()


def kernel(x):
    n = x.shape[0]
    total = 2 * n
    chunk = total // NUM_WORKERS
    sub = chunk // ROUNDS
    xf = x.reshape(n // 128, 128, 2).transpose(0, 2, 1).reshape(total)
    mesh = plsc.VectorSubcoreMesh(core_axis_name="c", subcore_axis_name="s")
    out = pl.kernel(
        functools.partial(_sc_body, chunk=chunk),
        out_type=jax.ShapeDtypeStruct((total,), jnp.float32),
        mesh=mesh,
        scratch_types=[
            pltpu.VMEM((sub,), jnp.float32),
            pltpu.VMEM((sub,), jnp.float32),
            pltpu.VMEM((sub,), jnp.float32),
            pltpu.VMEM((sub,), jnp.float32),
            pltpu.SemaphoreType.DMA((4,)),
        ],
    )(xf)
    return out.reshape(n // 128, 2, 128).transpose(0, 2, 1).reshape(n, 2)


# speedup vs baseline: 1.0071x; 1.0071x over previous
import functools

import jax
import jax.numpy as jnp
import numpy as np
from jax import lax
from jax.experimental import pallas as pl
from jax.experimental.pallas import tpu as pltpu
from jax.experimental.pallas import tpu_sc as plsc

SQRT3 = np.float32(3 ** 0.5)
HALF = np.float32(0.5)
MAGIC = np.float32(12582912.0)

NUM_WORKERS = 32
ROUNDS = 4


def _rne(v):
    return (v + MAGIC) - MAGIC


def _compute(src, dst, nvec):
    @plsc.parallel_loop(0, nvec, 1, unroll=4)
    def vec_body(v):
        off = lax.shift_left(v >> 3, 8) + lax.shift_left(v & 7, 4)
        a0 = src[pl.ds(off, 16)]
        a1 = src[pl.ds(off + 128, 16)]
        s1 = a1 / SQRT3
        r10 = _rne(a0)
        r11 = _rne(s1)
        r20 = _rne(a0 - HALF) + HALF
        r21 = _rne(s1 - HALF) + HALF
        y11 = r11 * SQRT3
        y21 = r21 * SQRT3
        e10 = a0 - r10
        e11 = a1 - y11
        e20 = a0 - r20
        e21 = a1 - y21
        d1 = e10 * e10 + e11 * e11
        d2 = e20 * e20 + e21 * e21
        take = d1 <= d2
        dst[pl.ds(off, 16)] = jnp.where(take, r10, r20)
        dst[pl.ds(off + 128, 16)] = jnp.where(take, y11, y21)


def _sc_body(x_hbm, o_hbm, in0, in1, out0, out1, sems, chunk):
    wid = lax.axis_index("c") * 16 + lax.axis_index("s")
    base = wid * chunk
    sub = chunk // ROUNDS
    nvec = sub // 32
    ins = (in0, in1)
    outs = (out0, out1)

    in_copies = [None] * ROUNDS
    out_copies = [None] * ROUNDS
    in_copies[0] = pltpu.async_copy(x_hbm.at[pl.ds(base, sub)], ins[0], sems.at[0])
    for r in range(ROUNDS):
        b = r % 2
        if r + 1 < ROUNDS:
            in_copies[r + 1] = pltpu.async_copy(
                x_hbm.at[pl.ds(base + (r + 1) * sub, sub)], ins[1 - b], sems.at[1 - b]
            )
        in_copies[r].wait()
        if r >= 2:
            out_copies[r - 2].wait()
        _compute(ins[b], outs[b], nvec)
        out_copies[r] = pltpu.async_copy(
            outs[b], o_hbm.at[pl.ds(base + r * sub, sub)], sems.at[2 + b]
        )
    out_copies[ROUNDS - 2].wait()
    out_copies[ROUNDS - 1].wait()


def kernel(x):
    n = x.shape[0]
    total = 2 * n
    chunk = total // NUM_WORKERS
    sub = chunk // ROUNDS
    xf = x.reshape(n // 128, 128, 2).transpose(0, 2, 1).reshape(total)
    mesh = plsc.VectorSubcoreMesh(core_axis_name="c", subcore_axis_name="s")
    out = pl.kernel(
        functools.partial(_sc_body, chunk=chunk),
        out_type=jax.ShapeDtypeStruct((total,), jnp.float32),
        mesh=mesh,
        scratch_types=[
            pltpu.VMEM((sub,), jnp.float32),
            pltpu.VMEM((sub,), jnp.float32),
            pltpu.VMEM((sub,), jnp.float32),
            pltpu.VMEM((sub,), jnp.float32),
            pltpu.SemaphoreType.DMA((4,)),
        ],
    )(xf)
    return out.reshape(n // 128, 2, 128).transpose(0, 2, 1).reshape(n, 2)
